# baseline (speedup 1.0000x reference)
import functools

import jax
import jax.numpy as jnp
from jax import lax
from jax.experimental import pallas as pl
from jax.experimental.pallas import tpu as pltpu
from jax.experimental.pallas import tpu_sc as plsc

N = 5120
NR = 5000
C = 128
NC = N // C
TIERS = 4
BT = NC // TIERS
OUTW = 512
TOPK = 400
TH = 0.65
F32 = jnp.float32
I32 = jnp.int32


def _dsm(off):
    return pl.ds(pl.multiple_of(off, C), C)


def _tcol(v):
    one = jnp.ones((1, 1), F32)
    return lax.dot_general(v, one, (((0,), (0,)), ((), ())),
                           preferred_element_type=F32,
                           precision=lax.Precision.HIGHEST)


def _nms_body(dataT_ref, rows_ref, scol_ref, out_ref, cnt_ref,
              sortedT_ref, srows_ref,
              x1c_ref, y1c_ref, x2c_ref, y2c_ref, ac_ref,
              keptc_ref, keptr_ref, posr_ref):
    srow = dataT_ref[4:5, :]
    irow = lax.broadcasted_iota(I32, (1, N), 1)

    def _rrow(c, acc):
        sj = scol_ref[_dsm(c * C), :]
        ij = lax.broadcasted_iota(I32, (C, 1), 0) + c * C
        before = (sj > srow) | ((sj == srow) & (ij < irow))
        return acc + jnp.sum(before.astype(F32), axis=0, keepdims=True)

    rank_r = lax.fori_loop(0, NC, _rrow, jnp.zeros((1, N), F32))

    rows = rows_ref[...]

    def _perm(c, _):
        d = _dsm(c * C)
        tc_ = (lax.broadcasted_iota(I32, (C, 1), 0) + c * C).astype(F32)
        ohT = (tc_ == rank_r).astype(F32)
        rc = jnp.dot(ohT, rows, preferred_element_type=F32,
                     precision=lax.Precision.HIGHEST)
        srows_ref[d, :] = rc
        x1 = rc[:, 0:1]
        y1 = rc[:, 1:2]
        x2 = rc[:, 2:3]
        y2 = rc[:, 3:4]
        x1c_ref[d, :] = x1
        y1c_ref[d, :] = y1
        x2c_ref[d, :] = x2
        y2c_ref[d, :] = y2
        ac_ref[d, :] = (x2 - x1) * (y2 - y1)
        return 0

    lax.fori_loop(0, NC, _perm, 0)

    eye8 = (lax.broadcasted_iota(I32, (8, 8), 0)
            == lax.broadcasted_iota(I32, (8, 8), 1)).astype(F32)
    sortedT_ref[...] = lax.dot_general(
        eye8, srows_ref[...], (((1,), (1,)), ((), ())),
        preferred_element_type=F32,
        precision=lax.Precision.HIGHEST)

    for t in range(TIERS):
        H = (t + 1) * (N // TIERS)

        def _blk(bl, _, t=t, H=H):
            b = t * BT + bl
            d = _dsm(b * C)
            bx1 = sortedT_ref[0:1, d]
            by1 = sortedT_ref[1:2, d]
            bx2 = sortedT_ref[2:3, d]
            by2 = sortedT_ref[3:4, d]
            ba = (bx2 - bx1) * (by2 - by1)
            ipos = lax.broadcasted_iota(I32, (1, C), 1) + b * C

            jx1 = x1c_ref[0:H, :]
            jy1 = y1c_ref[0:H, :]
            jx2 = x2c_ref[0:H, :]
            jy2 = y2c_ref[0:H, :]
            ja = ac_ref[0:H, :]
            jk = keptc_ref[0:H, :]
            jpos = lax.broadcasted_iota(I32, (H, 1), 0)
            w = jnp.maximum(jnp.minimum(jx2, bx2) - jnp.maximum(jx1, bx1),
                            0.0)
            h = jnp.maximum(jnp.minimum(jy2, by2) - jnp.maximum(jy1, by1),
                            0.0)
            inter = w * h
            union = ja + ba - inter
            hit = ((inter > TH * union) & (union >= 0.0)
                   & (jpos < b * C) & (jk > 0.5))
            ext = jnp.max(hit.astype(F32), axis=0, keepdims=True)

            lx1 = x1c_ref[d, :]
            ly1 = y1c_ref[d, :]
            lx2 = x2c_ref[d, :]
            ly2 = y2c_ref[d, :]
            la = ac_ref[d, :]
            jloc = lax.broadcasted_iota(I32, (C, C), 0)
            iloc = lax.broadcasted_iota(I32, (C, C), 1)
            w2 = jnp.maximum(jnp.minimum(lx2, bx2) - jnp.maximum(lx1, bx1),
                             0.0)
            h2 = jnp.maximum(jnp.minimum(ly2, by2) - jnp.maximum(ly1, by1),
                             0.0)
            inter2 = w2 * h2
            union2 = la + ba - inter2
            lhit = ((inter2 > TH * union2) & (union2 >= 0.0)
                    & (jloc < iloc)).astype(F32)

            realb = (ipos < NR).astype(F32)
            kb0 = (ext < 0.5).astype(F32) * realb

            def _lcond(st):
                return st[1]

            def _lbody(st):
                kb, _ = st
                cnt = jnp.dot(kb, lhit, preferred_element_type=F32)
                new = ((ext + cnt) < 0.5).astype(F32) * realb
                return (new, jnp.any(new != kb))

            kb, _ = lax.while_loop(_lcond, _lbody, (kb0, jnp.bool_(True)))
            keptr_ref[:, d] = kb
            keptc_ref[d, :] = _tcol(kb)
            return 0

        lax.fori_loop(0, BT, _blk, 0)

    tri = (lax.broadcasted_iota(I32, (C, C), 0)
           < lax.broadcasted_iota(I32, (C, C), 1)).astype(F32)

    def _pos(c, run):
        d = _dsm(c * C)
        kch = keptr_ref[:, d]
        posr_ref[:, d] = jnp.dot(kch, tri, preferred_element_type=F32) + run
        return run + jnp.sum(kch)

    count = lax.fori_loop(0, NC, _pos, jnp.float32(0.0))

    tgtc = lax.broadcasted_iota(I32, (OUTW, 1), 0).astype(F32)
    hsel = ((posr_ref[...] == tgtc)
            & (keptr_ref[...] > 0.5)).astype(F32)
    outv = jnp.dot(hsel, srows_ref[...], preferred_element_type=F32,
                   precision=lax.Precision.HIGHEST)
    out_ref[...] = jnp.where(tgtc < count, outv, -1.0)
    cnt_ref[...] = jnp.full((8, 128), count, F32)


SC_NW = 32
SC_RPW = OUTW // SC_NW
SC_D = 16


@functools.cache
def _sc_gather_fn():
    @functools.partial(
        pl.kernel,
        mesh=plsc.VectorSubcoreMesh(core_axis_name="c", subcore_axis_name="s"),
        compiler_params=pltpu.CompilerParams(use_tc_tiling_on_sc=False),
        out_type=jax.ShapeDtypeStruct((OUTW, SC_D), F32),
        scratch_types=[
            pltpu.VMEM((SC_RPW,), I32),
            pltpu.VMEM((SC_RPW, SC_D), F32),
            pltpu.VMEM((16,), I32),
            pltpu.SemaphoreType.DMA,
        ],
    )
    def _sc_gather(table_hbm, idx_hbm, cnt_hbm, out_hbm, idx_v, rows_v,
                   cnt_v, sem):
        wid = lax.axis_index("s") * 2 + lax.axis_index("c")
        base = wid * SC_RPW
        pltpu.sync_copy(idx_hbm.at[pl.ds(base, SC_RPW)], idx_v)
        pltpu.sync_copy(cnt_hbm, cnt_v)
        pltpu.async_copy(table_hbm.at[idx_v], rows_v, sem).wait()
        cnt = cnt_v[...]
        for r in range(SC_RPW):
            slot = jnp.zeros((16,), I32) + (base + r)
            rows_v[r] = jnp.where(slot < cnt, rows_v[r], -1.0)
        pltpu.sync_copy(rows_v, out_hbm.at[pl.ds(base, SC_RPW)])

    return _sc_gather


def kernel(detections):
    det = detections.astype(F32)
    rows = jnp.zeros((N, 8), F32)
    rows = rows.at[:, 4].set(-1.0)
    rows = rows.at[:, 5].set(jnp.arange(N, dtype=F32))
    rows = rows.at[:NR, :5].set(det)
    dataT = rows.T
    scol = rows[:, 4:5]
    outv, cntv = pl.pallas_call(
        _nms_body,
        out_shape=[jax.ShapeDtypeStruct((OUTW, 8), F32),
                   jax.ShapeDtypeStruct((8, 128), F32)],
        scratch_shapes=[
            pltpu.VMEM((8, N), F32),
            pltpu.VMEM((N, 8), F32),
            pltpu.VMEM((N, 1), F32),
            pltpu.VMEM((N, 1), F32),
            pltpu.VMEM((N, 1), F32),
            pltpu.VMEM((N, 1), F32),
            pltpu.VMEM((N, 1), F32),
            pltpu.VMEM((N, 1), F32),
            pltpu.VMEM((1, N), F32),
            pltpu.VMEM((1, N), F32),
        ],
    )(dataT, rows, scol)
    idx = jnp.maximum(outv[:, 5].astype(I32), 0)
    cnt16 = cntv[0, :16].astype(I32)
    table = jnp.zeros((N, SC_D), F32).at[:NR, :5].set(det)
    gathered = _sc_gather_fn()(table, idx, cnt16)
    return gathered[:TOPK, :5]


# speedup vs baseline: 1.1696x; 1.1696x over previous
import functools

import jax
import jax.numpy as jnp
from jax import lax
from jax.experimental import pallas as pl
from jax.experimental.pallas import tpu as pltpu
from jax.experimental.pallas import tpu_sc as plsc

N = 5120
NR = 5000
C = 128
NC = N // C
TIERS = 4
BT = NC // TIERS
OUTW = 512
TOPK = 400
TH = 0.65
F32 = jnp.float32
I32 = jnp.int32


def _dsm(off):
    return pl.ds(pl.multiple_of(off, C), C)


def _tcol(v):
    one = jnp.ones((1, 1), F32)
    return lax.dot_general(v, one, (((0,), (0,)), ((), ())),
                           preferred_element_type=F32)


def _nms_body(dataT_ref, rows_ref, scol_ref, out_ref, cnt_ref,
              sortedT_ref, srows_ref,
              x1c_ref, y1c_ref, x2c_ref, y2c_ref, ac_ref,
              keptc_ref, keptr_ref, posr_ref):
    srow = dataT_ref[4:5, :]
    irow = lax.broadcasted_iota(I32, (1, N), 1)

    def _rrow(c, acc):
        sj = scol_ref[_dsm(c * C), :]
        ij = lax.broadcasted_iota(I32, (C, 1), 0) + c * C
        before = (sj > srow) | ((sj == srow) & (ij < irow))
        return acc + jnp.sum(before.astype(F32), axis=0, keepdims=True)

    rank_r = lax.fori_loop(0, NC, _rrow, jnp.zeros((1, N), F32))

    rows24 = rows_ref[...]

    def _perm(c, _):
        d = _dsm(c * C)
        tc_ = (lax.broadcasted_iota(I32, (C, 1), 0) + c * C).astype(F32)
        ohT = (tc_ == rank_r).astype(F32)
        rc24 = jnp.dot(ohT, rows24, preferred_element_type=F32)
        rc = rc24[:, 0:8] + rc24[:, 8:16] + rc24[:, 16:24]
        srows_ref[d, :] = rc
        x1 = rc[:, 0:1]
        y1 = rc[:, 1:2]
        x2 = rc[:, 2:3]
        y2 = rc[:, 3:4]
        x1c_ref[d, :] = x1
        y1c_ref[d, :] = y1
        x2c_ref[d, :] = x2
        y2c_ref[d, :] = y2
        ac_ref[d, :] = (x2 - x1) * (y2 - y1)
        return 0

    lax.fori_loop(0, NC, _perm, 0)

    eye8 = (lax.broadcasted_iota(I32, (8, 8), 0)
            == lax.broadcasted_iota(I32, (8, 8), 1)).astype(F32)
    sortedT_ref[...] = lax.dot_general(
        eye8, srows_ref[...], (((1,), (1,)), ((), ())),
        preferred_element_type=F32,
        precision=lax.Precision.HIGHEST)

    for t in range(TIERS):
        H = (t + 1) * (N // TIERS)

        def _blk(bl, _, t=t, H=H):
            b = t * BT + bl
            d = _dsm(b * C)
            bx1 = sortedT_ref[0:1, d]
            by1 = sortedT_ref[1:2, d]
            bx2 = sortedT_ref[2:3, d]
            by2 = sortedT_ref[3:4, d]
            ba = (bx2 - bx1) * (by2 - by1)
            ipos = lax.broadcasted_iota(I32, (1, C), 1) + b * C

            jx1 = x1c_ref[0:H, :]
            jy1 = y1c_ref[0:H, :]
            jx2 = x2c_ref[0:H, :]
            jy2 = y2c_ref[0:H, :]
            ja = ac_ref[0:H, :]
            jk = keptc_ref[0:H, :]
            jpos = lax.broadcasted_iota(I32, (H, 1), 0)
            w = jnp.maximum(jnp.minimum(jx2, bx2) - jnp.maximum(jx1, bx1),
                            0.0)
            h = jnp.maximum(jnp.minimum(jy2, by2) - jnp.maximum(jy1, by1),
                            0.0)
            inter = w * h
            union = ja + ba - inter
            hit = ((inter > TH * union) & (union >= 0.0)
                   & (jpos < b * C) & (jk > 0.5))
            ext = jnp.max(hit.astype(F32), axis=0, keepdims=True)

            lx1 = x1c_ref[d, :]
            ly1 = y1c_ref[d, :]
            lx2 = x2c_ref[d, :]
            ly2 = y2c_ref[d, :]
            la = ac_ref[d, :]
            jloc = lax.broadcasted_iota(I32, (C, C), 0)
            iloc = lax.broadcasted_iota(I32, (C, C), 1)
            w2 = jnp.maximum(jnp.minimum(lx2, bx2) - jnp.maximum(lx1, bx1),
                             0.0)
            h2 = jnp.maximum(jnp.minimum(ly2, by2) - jnp.maximum(ly1, by1),
                             0.0)
            inter2 = w2 * h2
            union2 = la + ba - inter2
            lhit = ((inter2 > TH * union2) & (union2 >= 0.0)
                    & (jloc < iloc)).astype(F32)

            realb = (ipos < NR).astype(F32)
            kb0 = (ext < 0.5).astype(F32) * realb

            def _lcond(st):
                return st[1]

            def _lbody(st):
                kb, _ = st
                cnt = jnp.dot(kb, lhit, preferred_element_type=F32)
                new = ((ext + cnt) < 0.5).astype(F32) * realb
                return (new, jnp.any(new != kb))

            kb, _ = lax.while_loop(_lcond, _lbody, (kb0, jnp.bool_(True)))
            keptr_ref[:, d] = kb
            keptc_ref[d, :] = _tcol(kb)
            return 0

        lax.fori_loop(0, BT, _blk, 0)

    tri = (lax.broadcasted_iota(I32, (C, C), 0)
           < lax.broadcasted_iota(I32, (C, C), 1)).astype(F32)

    def _pos(c, run):
        d = _dsm(c * C)
        kch = keptr_ref[:, d]
        posr_ref[:, d] = jnp.dot(kch, tri, preferred_element_type=F32) + run
        return run + jnp.sum(kch)

    count = lax.fori_loop(0, NC, _pos, jnp.float32(0.0))

    tgtc = lax.broadcasted_iota(I32, (OUTW, 1), 0).astype(F32)
    hsel = ((posr_ref[...] == tgtc)
            & (keptr_ref[...] > 0.5)).astype(F32)
    outv = jnp.dot(hsel, srows_ref[...], preferred_element_type=F32)
    out_ref[...] = jnp.where(tgtc < count, outv, -1.0)
    cnt_ref[...] = jnp.full((8, 128), count, F32)


SC_NW = 32
SC_RPW = OUTW // SC_NW
SC_D = 16


@functools.cache
def _sc_gather_fn():
    @functools.partial(
        pl.kernel,
        mesh=plsc.VectorSubcoreMesh(core_axis_name="c", subcore_axis_name="s"),
        compiler_params=pltpu.CompilerParams(use_tc_tiling_on_sc=False),
        out_type=jax.ShapeDtypeStruct((OUTW, SC_D), F32),
        scratch_types=[
            pltpu.VMEM((SC_RPW,), I32),
            pltpu.VMEM((SC_RPW, SC_D), F32),
            pltpu.VMEM((16,), I32),
            pltpu.SemaphoreType.DMA,
        ],
    )
    def _sc_gather(table_hbm, idx_hbm, cnt_hbm, out_hbm, idx_v, rows_v,
                   cnt_v, sem):
        wid = lax.axis_index("s") * 2 + lax.axis_index("c")
        base = wid * SC_RPW
        pltpu.sync_copy(idx_hbm.at[pl.ds(base, SC_RPW)], idx_v)
        pltpu.sync_copy(cnt_hbm, cnt_v)
        pltpu.async_copy(table_hbm.at[idx_v], rows_v, sem).wait()
        cnt = cnt_v[...]
        for r in range(SC_RPW):
            slot = jnp.zeros((16,), I32) + (base + r)
            rows_v[r] = jnp.where(slot < cnt, rows_v[r], -1.0)
        pltpu.sync_copy(rows_v, out_hbm.at[pl.ds(base, SC_RPW)])

    return _sc_gather


def _split3(x):
    m = jnp.uint32(0xFFFF0000)
    xi = lax.bitcast_convert_type(x, jnp.uint32)
    b0 = lax.bitcast_convert_type(xi & m, F32)
    r1 = x - b0
    b1 = lax.bitcast_convert_type(lax.bitcast_convert_type(r1, jnp.uint32)
                                  & m, F32)
    b2 = r1 - b1
    return b0, b1, b2


def kernel(detections):
    det = detections.astype(F32)
    rows = jnp.zeros((N, 8), F32)
    rows = rows.at[:, 4].set(-1.0)
    idx_all = jnp.arange(N)
    rows = rows.at[:, 5].set((idx_all // 64).astype(F32))
    rows = rows.at[:, 6].set((idx_all % 64).astype(F32))
    rows = rows.at[:NR, :5].set(det)
    rows24 = jnp.concatenate(_split3(rows), axis=1)
    dataT = rows.T
    scol = rows[:, 4:5]
    outv, cntv = pl.pallas_call(
        _nms_body,
        out_shape=[jax.ShapeDtypeStruct((OUTW, 8), F32),
                   jax.ShapeDtypeStruct((8, 128), F32)],
        scratch_shapes=[
            pltpu.VMEM((8, N), F32),
            pltpu.VMEM((N, 8), F32),
            pltpu.VMEM((N, 1), F32),
            pltpu.VMEM((N, 1), F32),
            pltpu.VMEM((N, 1), F32),
            pltpu.VMEM((N, 1), F32),
            pltpu.VMEM((N, 1), F32),
            pltpu.VMEM((N, 1), F32),
            pltpu.VMEM((1, N), F32),
            pltpu.VMEM((1, N), F32),
        ],
    )(dataT, rows24, scol)
    idx = jnp.maximum(
        outv[:, 5].astype(I32) * 64 + outv[:, 6].astype(I32),
        0)
    cnt16 = cntv[0, :16].astype(I32)
    table = jnp.zeros((N, SC_D), F32).at[:NR, :5].set(det)
    gathered = _sc_gather_fn()(table, idx, cnt16)
    return gathered[:TOPK, :5]


# speedup vs baseline: 1.2694x; 1.0853x over previous
import functools

import jax
import jax.numpy as jnp
from jax import lax
from jax.experimental import pallas as pl
from jax.experimental.pallas import tpu as pltpu
from jax.experimental.pallas import tpu_sc as plsc

N = 5120
NR = 5000
C = 128
NC = N // C
TIERS = 4
BT = NC // TIERS
OUTW = 512
TOPK = 400
TH = 0.65
ONE_TH = 1.0 + 0.65
F32 = jnp.float32
I32 = jnp.int32


def _dsm(off):
    return pl.ds(pl.multiple_of(off, C), C)


def _tcol(v):
    one = jnp.ones((1, 1), F32)
    return lax.dot_general(v, one, (((0,), (0,)), ((), ())),
                           preferred_element_type=F32)


def _nms_body(dataT_ref, rows_ref, scol_ref, out_ref,
              sortedT_ref, srows_ref,
              x1c_ref, y1c_ref, x2c_ref, y2c_ref, ac_ref,
              keptc_ref, keptr_ref, posr_ref):
    srow = dataT_ref[4:5, :]
    irow = lax.broadcasted_iota(I32, (1, N), 1)

    onesC = jnp.ones((1, C), F32)

    def _rrow(c, acc):
        sj = scol_ref[_dsm(c * C), :]
        ij = lax.broadcasted_iota(I32, (C, 1), 0) + c * C
        before = ((sj > srow) | ((sj == srow) & (ij < irow))).astype(F32)
        return acc + jnp.dot(onesC, before, preferred_element_type=F32)

    rank_r = lax.fori_loop(0, NC, _rrow, jnp.zeros((1, N), F32))

    rows24 = rows_ref[...]

    def _perm(c, _):
        d = _dsm(c * C)
        tc_ = (lax.broadcasted_iota(I32, (C, 1), 0) + c * C).astype(F32)
        ohT = (tc_ == rank_r).astype(F32)
        rc24 = jnp.dot(ohT, rows24, preferred_element_type=F32)
        rc = rc24[:, 0:8] + rc24[:, 8:16] + rc24[:, 16:24]
        srows_ref[d, :] = rc
        x1 = rc[:, 0:1]
        y1 = rc[:, 1:2]
        x2 = rc[:, 2:3]
        y2 = rc[:, 3:4]
        x1c_ref[d, :] = x1
        y1c_ref[d, :] = y1
        x2c_ref[d, :] = x2
        y2c_ref[d, :] = y2
        ac_ref[d, :] = (x2 - x1) * (y2 - y1)
        return 0

    lax.fori_loop(0, NC, _perm, 0)

    eye8 = (lax.broadcasted_iota(I32, (8, 8), 0)
            == lax.broadcasted_iota(I32, (8, 8), 1)).astype(F32)
    sortedT_ref[...] = lax.dot_general(
        eye8, srows_ref[...], (((1,), (1,)), ((), ())),
        preferred_element_type=F32,
        precision=lax.Precision.HIGHEST)

    tri_b = (lax.broadcasted_iota(I32, (C, C), 0)
             < lax.broadcasted_iota(I32, (C, C), 1))
    tri = tri_b.astype(F32)
    keptc_ref[...] = jnp.zeros((N, 1), F32)

    for t in range(TIERS):
        H = (t + 1) * (N // TIERS)

        def _blk(bl, _, t=t, H=H):
            b = t * BT + bl
            d = _dsm(b * C)
            bx1 = sortedT_ref[0:1, d]
            by1 = sortedT_ref[1:2, d]
            bx2 = sortedT_ref[2:3, d]
            by2 = sortedT_ref[3:4, d]
            ba = (bx2 - bx1) * (by2 - by1)
            ipos = lax.broadcasted_iota(I32, (1, C), 1) + b * C

            jx1 = x1c_ref[0:H, :]
            jy1 = y1c_ref[0:H, :]
            jx2 = x2c_ref[0:H, :]
            jy2 = y2c_ref[0:H, :]
            ja = ac_ref[0:H, :]
            jk = keptc_ref[0:H, :]
            w = jnp.maximum(jnp.minimum(jx2, bx2) - jnp.maximum(jx1, bx1),
                            0.0)
            h = jnp.maximum(jnp.minimum(jy2, by2) - jnp.maximum(jy1, by1),
                            0.0)
            inter = w * h
            asum = ja + ba
            hit = ((ONE_TH * inter > TH * asum)
                   & (asum >= inter)).astype(F32)
            cnts = lax.dot_general(jk, hit, (((0,), (0,)), ((), ())),
                                   preferred_element_type=F32)
            ext = (cnts > 0.5).astype(F32)

            lx1 = x1c_ref[d, :]
            ly1 = y1c_ref[d, :]
            lx2 = x2c_ref[d, :]
            ly2 = y2c_ref[d, :]
            la = ac_ref[d, :]
            w2 = jnp.maximum(jnp.minimum(lx2, bx2) - jnp.maximum(lx1, bx1),
                             0.0)
            h2 = jnp.maximum(jnp.minimum(ly2, by2) - jnp.maximum(ly1, by1),
                             0.0)
            inter2 = w2 * h2
            asum2 = la + ba
            lhit = ((ONE_TH * inter2 > TH * asum2) & (asum2 >= inter2)
                    & tri_b).astype(F32)

            realb = (ipos < NR).astype(F32)
            kb0 = (1.0 - ext) * realb

            def _lcond(st):
                return st[1]

            def _lbody(st):
                kb, _ = st
                cnt = jnp.dot(kb, lhit, preferred_element_type=F32)
                new = ((ext + cnt) < 0.5).astype(F32) * realb
                return (new, jnp.any(new != kb))

            kb, _ = lax.while_loop(_lcond, _lbody, (kb0, jnp.bool_(True)))
            keptr_ref[:, d] = kb
            keptc_ref[d, :] = _tcol(kb)
            return 0

        lax.fori_loop(0, BT, _blk, 0)

    def _pos(c, run):
        d = _dsm(c * C)
        kch = keptr_ref[:, d]
        posr_ref[:, d] = jnp.dot(kch, tri, preferred_element_type=F32) + run
        return run + jnp.sum(kch)

    count = lax.fori_loop(0, NC, _pos, jnp.float32(0.0))

    tgtc = lax.broadcasted_iota(I32, (OUTW, 1), 0).astype(F32)
    hsel = ((posr_ref[...] == tgtc)
            & (keptr_ref[...] > 0.5)).astype(F32)
    outv = jnp.dot(hsel, srows_ref[...], preferred_element_type=F32)
    out_ref[...] = jnp.where(tgtc < count, outv, -1.0)


SC_NW = 32
SC_RPW = OUTW // SC_NW
SC_D = 16


@functools.cache
def _sc_gather_fn():
    @functools.partial(
        pl.kernel,
        mesh=plsc.VectorSubcoreMesh(core_axis_name="c", subcore_axis_name="s"),
        compiler_params=pltpu.CompilerParams(use_tc_tiling_on_sc=False),
        out_type=jax.ShapeDtypeStruct((OUTW, SC_D), F32),
        scratch_types=[
            pltpu.VMEM((SC_RPW,), I32),
            pltpu.VMEM((SC_RPW, SC_D), F32),
            pltpu.SemaphoreType.DMA,
        ],
    )
    def _sc_gather(table_hbm, idx_hbm, out_hbm, idx_v, rows_v, sem):
        wid = lax.axis_index("s") * 2 + lax.axis_index("c")
        base = wid * SC_RPW
        pltpu.sync_copy(idx_hbm.at[pl.ds(base, SC_RPW)], idx_v)
        pltpu.async_copy(table_hbm.at[idx_v], rows_v, sem).wait()
        pltpu.sync_copy(rows_v, out_hbm.at[pl.ds(base, SC_RPW)])

    return _sc_gather


def _split3(x):
    m = jnp.uint32(0xFFFF0000)
    xi = lax.bitcast_convert_type(x, jnp.uint32)
    b0 = lax.bitcast_convert_type(xi & m, F32)
    r1 = x - b0
    b1 = lax.bitcast_convert_type(lax.bitcast_convert_type(r1, jnp.uint32)
                                  & m, F32)
    b2 = r1 - b1
    return b0, b1, b2


def kernel(detections):
    det = detections.astype(F32)
    rows = jnp.zeros((N, 8), F32)
    rows = rows.at[:, 4].set(-1.0)
    idx_all = jnp.arange(N)
    rows = rows.at[:, 5].set((idx_all // 64).astype(F32))
    rows = rows.at[:, 6].set((idx_all % 64).astype(F32))
    rows = rows.at[:NR, :5].set(det)
    rows24 = jnp.concatenate(_split3(rows), axis=1)
    dataT = rows.T
    scol = rows[:, 4:5]
    outv = pl.pallas_call(
        _nms_body,
        out_shape=jax.ShapeDtypeStruct((OUTW, 8), F32),
        scratch_shapes=[
            pltpu.VMEM((8, N), F32),
            pltpu.VMEM((N, 8), F32),
            pltpu.VMEM((N, 1), F32),
            pltpu.VMEM((N, 1), F32),
            pltpu.VMEM((N, 1), F32),
            pltpu.VMEM((N, 1), F32),
            pltpu.VMEM((N, 1), F32),
            pltpu.VMEM((N, 1), F32),
            pltpu.VMEM((1, N), F32),
            pltpu.VMEM((1, N), F32),
        ],
    )(dataT, rows24, scol)
    idx_raw = outv[:, 5].astype(I32) * 64 + outv[:, 6].astype(I32)
    idx = jnp.where(idx_raw < 0, N, idx_raw)
    table = jnp.zeros((N + 8, SC_D), F32).at[:NR, :5].set(det)
    table = table.at[N:].set(-1.0)
    gathered = _sc_gather_fn()(table, idx)
    return gathered[:TOPK, :5]


# speedup vs baseline: 1.7119x; 1.3487x over previous
import functools

import jax
import jax.numpy as jnp
from jax import lax
from jax.experimental import pallas as pl
from jax.experimental.pallas import tpu as pltpu
from jax.experimental.pallas import tpu_sc as plsc

N = 5120
NR = 5000
C = 256
NC = N // C
TIERS = 10
BT = NC // TIERS
OUTW = 512
TOPK = 400
TH = 0.65
ONE_TH = 1.0 + 0.65
F32 = jnp.float32
I32 = jnp.int32


def _dsm(off):
    return pl.ds(pl.multiple_of(off, C), C)


def _tcol(v):
    one = jnp.ones((1, 1), F32)
    return lax.dot_general(v, one, (((0,), (0,)), ((), ())),
                           preferred_element_type=F32)


def _nms_body(dataT_ref, rows_ref, scol_ref, out_ref,
              sortedT_ref, srows_ref,
              x1c_ref, y1c_ref, x2c_ref, y2c_ref, ac_ref,
              keptc_ref, keptr_ref, posr_ref):
    srow = dataT_ref[4:5, :]
    irow = lax.broadcasted_iota(I32, (1, N), 1)

    onesC = jnp.ones((1, C), F32)

    def _rrow(c, acc):
        sj = scol_ref[_dsm(c * C), :]
        ij = lax.broadcasted_iota(I32, (C, 1), 0) + c * C
        before = ((sj > srow) | ((sj == srow) & (ij < irow))).astype(F32)
        return acc + jnp.dot(onesC, before, preferred_element_type=F32)

    rank_r = lax.fori_loop(0, NC, _rrow, jnp.zeros((1, N), F32))

    rows24 = rows_ref[...]

    def _perm(c, _):
        d = _dsm(c * C)
        tc_ = (lax.broadcasted_iota(I32, (C, 1), 0) + c * C).astype(F32)
        ohT = (tc_ == rank_r).astype(F32)
        rc24 = jnp.dot(ohT, rows24, preferred_element_type=F32)
        rc = rc24[:, 0:8] + rc24[:, 8:16] + rc24[:, 16:24]
        srows_ref[d, :] = rc
        x1 = rc[:, 0:1]
        y1 = rc[:, 1:2]
        x2 = rc[:, 2:3]
        y2 = rc[:, 3:4]
        x1c_ref[d, :] = x1
        y1c_ref[d, :] = y1
        x2c_ref[d, :] = x2
        y2c_ref[d, :] = y2
        ac_ref[d, :] = (x2 - x1) * (y2 - y1)
        return 0

    lax.fori_loop(0, NC, _perm, 0)

    eye8 = (lax.broadcasted_iota(I32, (8, 8), 0)
            == lax.broadcasted_iota(I32, (8, 8), 1)).astype(F32)
    sortedT_ref[...] = lax.dot_general(
        eye8, srows_ref[...], (((1,), (1,)), ((), ())),
        preferred_element_type=F32,
        precision=lax.Precision.HIGHEST)

    tri_b = (lax.broadcasted_iota(I32, (C, C), 0)
             < lax.broadcasted_iota(I32, (C, C), 1))
    tri = tri_b.astype(F32)
    keptc_ref[...] = jnp.zeros((N, 1), F32)

    for t in range(TIERS):
        H = (t + 1) * (N // TIERS)

        def _blk(bl, _, t=t, H=H):
            b = t * BT + bl
            d = _dsm(b * C)
            bx1 = sortedT_ref[0:1, d]
            by1 = sortedT_ref[1:2, d]
            bx2 = sortedT_ref[2:3, d]
            by2 = sortedT_ref[3:4, d]
            ba = (bx2 - bx1) * (by2 - by1)
            ipos = lax.broadcasted_iota(I32, (1, C), 1) + b * C

            jx1 = x1c_ref[0:H, :]
            jy1 = y1c_ref[0:H, :]
            jx2 = x2c_ref[0:H, :]
            jy2 = y2c_ref[0:H, :]
            ja = ac_ref[0:H, :]
            jk = keptc_ref[0:H, :]
            w = jnp.maximum(jnp.minimum(jx2, bx2) - jnp.maximum(jx1, bx1),
                            0.0)
            h = jnp.maximum(jnp.minimum(jy2, by2) - jnp.maximum(jy1, by1),
                            0.0)
            inter = w * h
            asum = ja + ba
            hit = ((ONE_TH * inter > TH * asum)
                   & (asum >= inter)).astype(F32)
            cnts = lax.dot_general(jk, hit, (((0,), (0,)), ((), ())),
                                   preferred_element_type=F32)
            ext = (cnts > 0.5).astype(F32)

            lx1 = x1c_ref[d, :]
            ly1 = y1c_ref[d, :]
            lx2 = x2c_ref[d, :]
            ly2 = y2c_ref[d, :]
            la = ac_ref[d, :]
            w2 = jnp.maximum(jnp.minimum(lx2, bx2) - jnp.maximum(lx1, bx1),
                             0.0)
            h2 = jnp.maximum(jnp.minimum(ly2, by2) - jnp.maximum(ly1, by1),
                             0.0)
            inter2 = w2 * h2
            asum2 = la + ba
            lhit = ((ONE_TH * inter2 > TH * asum2) & (asum2 >= inter2)
                    & tri_b).astype(F32)

            realb = (ipos < NR).astype(F32)
            kb0 = (1.0 - ext) * realb

            def _lcond(st):
                return st[1]

            def _lbody(st):
                kb, _ = st
                cnt = jnp.dot(kb, lhit, preferred_element_type=F32)
                new = ((ext + cnt) < 0.5).astype(F32) * realb
                return (new, jnp.any(new != kb))

            kb, _ = lax.while_loop(_lcond, _lbody, (kb0, jnp.bool_(True)))
            keptr_ref[:, d] = kb
            keptc_ref[d, :] = _tcol(kb)
            return 0

        lax.fori_loop(0, BT, _blk, 0)

    def _pos(c, run):
        d = _dsm(c * C)
        kch = keptr_ref[:, d]
        posr_ref[:, d] = jnp.dot(kch, tri, preferred_element_type=F32) + run
        return run + jnp.sum(kch)

    count = lax.fori_loop(0, NC, _pos, jnp.float32(0.0))

    tgtc = lax.broadcasted_iota(I32, (OUTW, 1), 0).astype(F32)
    hsel = ((posr_ref[...] == tgtc)
            & (keptr_ref[...] > 0.5)).astype(F32)
    outv = jnp.dot(hsel, srows_ref[...], preferred_element_type=F32)
    out_ref[...] = jnp.where(tgtc < count, outv, -1.0)


SC_NW = 32
SC_RPW = OUTW // SC_NW
SC_D = 16


@functools.cache
def _sc_gather_fn():
    @functools.partial(
        pl.kernel,
        mesh=plsc.VectorSubcoreMesh(core_axis_name="c", subcore_axis_name="s"),
        compiler_params=pltpu.CompilerParams(use_tc_tiling_on_sc=False),
        out_type=jax.ShapeDtypeStruct((OUTW, SC_D), F32),
        scratch_types=[
            pltpu.VMEM((SC_RPW,), I32),
            pltpu.VMEM((SC_RPW, SC_D), F32),
            pltpu.SemaphoreType.DMA,
        ],
    )
    def _sc_gather(table_hbm, idx_hbm, out_hbm, idx_v, rows_v, sem):
        wid = lax.axis_index("s") * 2 + lax.axis_index("c")
        base = wid * SC_RPW
        pltpu.sync_copy(idx_hbm.at[pl.ds(base, SC_RPW)], idx_v)
        pltpu.async_copy(table_hbm.at[idx_v], rows_v, sem).wait()
        pltpu.sync_copy(rows_v, out_hbm.at[pl.ds(base, SC_RPW)])

    return _sc_gather


def _split3(x):
    m = jnp.uint32(0xFFFF0000)
    xi = lax.bitcast_convert_type(x, jnp.uint32)
    b0 = lax.bitcast_convert_type(xi & m, F32)
    r1 = x - b0
    b1 = lax.bitcast_convert_type(lax.bitcast_convert_type(r1, jnp.uint32)
                                  & m, F32)
    b2 = r1 - b1
    return b0, b1, b2


def kernel(detections):
    det = detections.astype(F32)
    rows = jnp.zeros((N, 8), F32)
    rows = rows.at[:, 4].set(-1.0)
    idx_all = jnp.arange(N)
    rows = rows.at[:, 5].set((idx_all // 64).astype(F32))
    rows = rows.at[:, 6].set((idx_all % 64).astype(F32))
    rows = rows.at[:NR, :5].set(det)
    rows24 = jnp.concatenate(_split3(rows), axis=1)
    dataT = rows.T
    scol = rows[:, 4:5]
    outv = pl.pallas_call(
        _nms_body,
        out_shape=jax.ShapeDtypeStruct((OUTW, 8), F32),
        scratch_shapes=[
            pltpu.VMEM((8, N), F32),
            pltpu.VMEM((N, 8), F32),
            pltpu.VMEM((N, 1), F32),
            pltpu.VMEM((N, 1), F32),
            pltpu.VMEM((N, 1), F32),
            pltpu.VMEM((N, 1), F32),
            pltpu.VMEM((N, 1), F32),
            pltpu.VMEM((N, 1), F32),
            pltpu.VMEM((1, N), F32),
            pltpu.VMEM((1, N), F32),
        ],
    )(dataT, rows24, scol)
    idx_raw = outv[:, 5].astype(I32) * 64 + outv[:, 6].astype(I32)
    idx = jnp.where(idx_raw < 0, N, idx_raw)
    table = jnp.zeros((N + 8, SC_D), F32).at[:NR, :5].set(det)
    table = table.at[N:].set(-1.0)
    gathered = _sc_gather_fn()(table, idx)
    return gathered[:TOPK, :5]


# speedup vs baseline: 1.9138x; 1.1179x over previous
import functools

import jax
import jax.numpy as jnp
from jax import lax
from jax.experimental import pallas as pl
from jax.experimental.pallas import tpu as pltpu
from jax.experimental.pallas import tpu_sc as plsc

N = 5120
NR = 5000
C = 512
NC = N // C
TIERS = 10
BT = NC // TIERS
OUTW = 512
TOPK = 400
TH = 0.65
ONE_TH = 1.0 + 0.65
F32 = jnp.float32
I32 = jnp.int32


def _dsm(off):
    return pl.ds(pl.multiple_of(off, C), C)


def _tcol(v):
    one = jnp.ones((1, 1), F32)
    return lax.dot_general(v, one, (((0,), (0,)), ((), ())),
                           preferred_element_type=F32)


def _nms_body(dataT_ref, rows_ref, scol_ref, out_ref,
              sortedT_ref, srows_ref,
              x1c_ref, y1c_ref, x2c_ref, y2c_ref, ac_ref,
              keptc_ref, keptr_ref, posr_ref):
    srow = dataT_ref[4:5, :]
    irow = lax.broadcasted_iota(I32, (1, N), 1)

    onesC = jnp.ones((1, C), F32)

    def _rrow(c, acc):
        sj = scol_ref[_dsm(c * C), :]
        ij = lax.broadcasted_iota(I32, (C, 1), 0) + c * C
        before = ((sj > srow) | ((sj == srow) & (ij < irow))).astype(F32)
        return acc + jnp.dot(onesC, before, preferred_element_type=F32)

    rank_r = lax.fori_loop(0, NC, _rrow, jnp.zeros((1, N), F32))

    rows24 = rows_ref[...]

    def _perm(c, _):
        d = _dsm(c * C)
        tc_ = (lax.broadcasted_iota(I32, (C, 1), 0) + c * C).astype(F32)
        ohT = (tc_ == rank_r).astype(F32)
        rc24 = jnp.dot(ohT, rows24, preferred_element_type=F32)
        rc = rc24[:, 0:8] + rc24[:, 8:16] + rc24[:, 16:24]
        srows_ref[d, :] = rc
        x1 = rc[:, 0:1]
        y1 = rc[:, 1:2]
        x2 = rc[:, 2:3]
        y2 = rc[:, 3:4]
        x1c_ref[d, :] = x1
        y1c_ref[d, :] = y1
        x2c_ref[d, :] = x2
        y2c_ref[d, :] = y2
        ac_ref[d, :] = (x2 - x1) * (y2 - y1)
        return 0

    lax.fori_loop(0, NC, _perm, 0)

    eye8 = (lax.broadcasted_iota(I32, (8, 8), 0)
            == lax.broadcasted_iota(I32, (8, 8), 1)).astype(F32)
    sortedT_ref[...] = lax.dot_general(
        eye8, srows_ref[...], (((1,), (1,)), ((), ())),
        preferred_element_type=F32,
        precision=lax.Precision.HIGHEST)

    tri_b = (lax.broadcasted_iota(I32, (C, C), 0)
             < lax.broadcasted_iota(I32, (C, C), 1))
    tri = tri_b.astype(F32)
    keptc_ref[...] = jnp.zeros((N, 1), F32)

    for t in range(TIERS):
        H = (t + 1) * (N // TIERS)

        def _blk(bl, _, t=t, H=H):
            b = t * BT + bl
            d = _dsm(b * C)
            bx1 = sortedT_ref[0:1, d]
            by1 = sortedT_ref[1:2, d]
            bx2 = sortedT_ref[2:3, d]
            by2 = sortedT_ref[3:4, d]
            ba = (bx2 - bx1) * (by2 - by1)
            ipos = lax.broadcasted_iota(I32, (1, C), 1) + b * C

            jx1 = x1c_ref[0:H, :]
            jy1 = y1c_ref[0:H, :]
            jx2 = x2c_ref[0:H, :]
            jy2 = y2c_ref[0:H, :]
            ja = ac_ref[0:H, :]
            jk = keptc_ref[0:H, :]
            w = jnp.maximum(jnp.minimum(jx2, bx2) - jnp.maximum(jx1, bx1),
                            0.0)
            h = jnp.maximum(jnp.minimum(jy2, by2) - jnp.maximum(jy1, by1),
                            0.0)
            inter = w * h
            asum = ja + ba
            hit = ((ONE_TH * inter > TH * asum)
                   & (asum >= inter)).astype(F32)
            cnts = lax.dot_general(jk, hit, (((0,), (0,)), ((), ())),
                                   preferred_element_type=F32)
            ext = (cnts > 0.5).astype(F32)

            lx1 = x1c_ref[d, :]
            ly1 = y1c_ref[d, :]
            lx2 = x2c_ref[d, :]
            ly2 = y2c_ref[d, :]
            la = ac_ref[d, :]
            w2 = jnp.maximum(jnp.minimum(lx2, bx2) - jnp.maximum(lx1, bx1),
                             0.0)
            h2 = jnp.maximum(jnp.minimum(ly2, by2) - jnp.maximum(ly1, by1),
                             0.0)
            inter2 = w2 * h2
            asum2 = la + ba
            lhit = ((ONE_TH * inter2 > TH * asum2) & (asum2 >= inter2)
                    & tri_b).astype(F32)

            realb = (ipos < NR).astype(F32)
            kb0 = (1.0 - ext) * realb

            def _lcond(st):
                return st[1]

            def _lbody(st):
                kb, _ = st
                cnt = jnp.dot(kb, lhit, preferred_element_type=F32)
                new = ((ext + cnt) < 0.5).astype(F32) * realb
                return (new, jnp.any(new != kb))

            kb, _ = lax.while_loop(_lcond, _lbody, (kb0, jnp.bool_(True)))
            keptr_ref[:, d] = kb
            keptc_ref[d, :] = _tcol(kb)
            return 0

        lax.fori_loop(0, BT, _blk, 0)

    def _pos(c, run):
        d = _dsm(c * C)
        kch = keptr_ref[:, d]
        posr_ref[:, d] = jnp.dot(kch, tri, preferred_element_type=F32) + run
        return run + jnp.sum(kch)

    count = lax.fori_loop(0, NC, _pos, jnp.float32(0.0))

    tgtc = lax.broadcasted_iota(I32, (OUTW, 1), 0).astype(F32)
    hsel = ((posr_ref[...] == tgtc)
            & (keptr_ref[...] > 0.5)).astype(F32)
    outv = jnp.dot(hsel, srows_ref[...], preferred_element_type=F32)
    out_ref[...] = jnp.where(tgtc < count, outv, -1.0)


SC_NW = 32
SC_RPW = OUTW // SC_NW
SC_D = 16


@functools.cache
def _sc_gather_fn():
    @functools.partial(
        pl.kernel,
        mesh=plsc.VectorSubcoreMesh(core_axis_name="c", subcore_axis_name="s"),
        compiler_params=pltpu.CompilerParams(use_tc_tiling_on_sc=False),
        out_type=jax.ShapeDtypeStruct((OUTW, SC_D), F32),
        scratch_types=[
            pltpu.VMEM((SC_RPW,), I32),
            pltpu.VMEM((SC_RPW, SC_D), F32),
            pltpu.SemaphoreType.DMA,
        ],
    )
    def _sc_gather(table_hbm, idx_hbm, out_hbm, idx_v, rows_v, sem):
        wid = lax.axis_index("s") * 2 + lax.axis_index("c")
        base = wid * SC_RPW
        pltpu.sync_copy(idx_hbm.at[pl.ds(base, SC_RPW)], idx_v)
        pltpu.async_copy(table_hbm.at[idx_v], rows_v, sem).wait()
        pltpu.sync_copy(rows_v, out_hbm.at[pl.ds(base, SC_RPW)])

    return _sc_gather


def _split3(x):
    m = jnp.uint32(0xFFFF0000)
    xi = lax.bitcast_convert_type(x, jnp.uint32)
    b0 = lax.bitcast_convert_type(xi & m, F32)
    r1 = x - b0
    b1 = lax.bitcast_convert_type(lax.bitcast_convert_type(r1, jnp.uint32)
                                  & m, F32)
    b2 = r1 - b1
    return b0, b1, b2


def kernel(detections):
    det = detections.astype(F32)
    rows = jnp.zeros((N, 8), F32)
    rows = rows.at[:, 4].set(-1.0)
    idx_all = jnp.arange(N)
    rows = rows.at[:, 5].set((idx_all // 64).astype(F32))
    rows = rows.at[:, 6].set((idx_all % 64).astype(F32))
    rows = rows.at[:NR, :5].set(det)
    rows24 = jnp.concatenate(_split3(rows), axis=1)
    dataT = rows.T
    scol = rows[:, 4:5]
    outv = pl.pallas_call(
        _nms_body,
        out_shape=jax.ShapeDtypeStruct((OUTW, 8), F32),
        scratch_shapes=[
            pltpu.VMEM((8, N), F32),
            pltpu.VMEM((N, 8), F32),
            pltpu.VMEM((N, 1), F32),
            pltpu.VMEM((N, 1), F32),
            pltpu.VMEM((N, 1), F32),
            pltpu.VMEM((N, 1), F32),
            pltpu.VMEM((N, 1), F32),
            pltpu.VMEM((N, 1), F32),
            pltpu.VMEM((1, N), F32),
            pltpu.VMEM((1, N), F32),
        ],
    )(dataT, rows24, scol)
    idx_raw = outv[:, 5].astype(I32) * 64 + outv[:, 6].astype(I32)
    idx = jnp.where(idx_raw < 0, N, idx_raw)
    table = jnp.zeros((N + 8, SC_D), F32).at[:NR, :5].set(det)
    table = table.at[N:].set(-1.0)
    gathered = _sc_gather_fn()(table, idx)
    return gathered[:TOPK, :5]
